# Initial kernel scaffold; baseline (speedup 1.0000x reference)
import jax
import jax.numpy as jnp
from jax.experimental import pallas as pl


def kernel(activations, cache, n_valid, index):
    raise NotImplementedError("write your pallas kernel here")


# speedup vs baseline: 3.3743x; 3.3743x over previous
import jax
import jax.numpy as jnp
from jax import lax
from jax.experimental import pallas as pl
from jax.experimental.pallas import tpu as pltpu
from jax.experimental.pallas import tpu_sc as plsc

DP = 2
MAX_SAMPLES = 16384
N_DIM = 1024
NW = 32
CH = 32
TOTAL_ROWS = DP * MAX_SAMPLES
ROWS_PER_W = TOTAL_ROWS // NW
BPW = ROWS_PER_W // CH
NBLK = TOTAL_ROWS // CH
ACTS_FLAG = 1 << 20


def _build_table(index, chunk):
    b = jnp.arange(NBLK, dtype=jnp.int32)
    r0 = b * CH
    d = r0 // MAX_SAMPLES
    rdp = r0 % MAX_SAMPLES
    off = (rdp - index) % MAX_SAMPLES
    in_acts = off < chunk
    src_acts = jnp.minimum(d * chunk + off, DP * chunk - CH) + ACTS_FLAG
    src = jnp.where(in_acts, src_acts, r0).astype(jnp.int32)
    return jnp.broadcast_to(src[:, None], (NBLK, 16))


def _copy_body(acts_hbm, cache_hbm, tbl_hbm, out_hbm, tbl_v, buf):
    wid = lax.axis_index("c") * 16 + lax.axis_index("s")
    base = pl.multiple_of(wid * ROWS_PER_W, CH)
    pltpu.sync_copy(tbl_hbm.at[pl.ds(pl.multiple_of(wid * BPW, BPW), BPW)],
                    tbl_v)

    def body(i, _):
        s = tbl_v[i][0]
        is_acts = s >= ACTS_FLAG

        @pl.when(is_acts)
        def _():
            pltpu.sync_copy(
                acts_hbm.at[pl.ds(pl.multiple_of(s - ACTS_FLAG, 8), CH)], buf)

        @pl.when(jnp.logical_not(is_acts))
        def _():
            pltpu.sync_copy(cache_hbm.at[pl.ds(pl.multiple_of(s, 8), CH)],
                            buf)

        pltpu.sync_copy(buf,
                        out_hbm.at[pl.ds(pl.multiple_of(base + i * CH, CH),
                                         CH)])
        return 0

    lax.fori_loop(0, BPW, body, 0)


def kernel(activations, cache, n_valid, index):
    dp, max_samples, d = cache.shape
    acts = activations.reshape((dp, -1, d))
    chunk = acts.shape[1]
    new_n_valid = jnp.minimum(jnp.asarray(n_valid) + chunk, max_samples)
    new_index = (jnp.asarray(index) + chunk) % max_samples

    acts_flat = activations.astype(cache.dtype)
    cache_flat = cache.reshape((dp * max_samples, d))
    tbl = _build_table(jnp.asarray(index, dtype=jnp.int32), chunk)

    mesh = plsc.VectorSubcoreMesh(core_axis_name="c", subcore_axis_name="s")
    out_flat = pl.kernel(
        _copy_body,
        mesh=mesh,
        out_type=jax.ShapeDtypeStruct((dp * max_samples, d), cache.dtype),
        scratch_types=[
            pltpu.VMEM((BPW, 16), jnp.int32),
            pltpu.VMEM((CH, N_DIM), jnp.float32),
        ],
    )(acts_flat, cache_flat, tbl)

    new_cache = out_flat.reshape((dp, max_samples, d))
    return (new_cache, new_n_valid, new_index)


# speedup vs baseline: 3.9757x; 1.1782x over previous
import jax
import jax.numpy as jnp
from jax import lax
from jax.experimental import pallas as pl
from jax.experimental.pallas import tpu as pltpu
from jax.experimental.pallas import tpu_sc as plsc

DP = 2
MAX_SAMPLES = 16384
N_DIM = 1024
NW = 32
CH = 32
TOTAL_ROWS = DP * MAX_SAMPLES
ROWS_PER_W = TOTAL_ROWS // NW
BPW = ROWS_PER_W // CH
NBLK = TOTAL_ROWS // CH
ACTS_FLAG = 1 << 20


def _build_table(index, chunk):
    b = jnp.arange(NBLK, dtype=jnp.int32)
    r0 = b * CH
    d = r0 // MAX_SAMPLES
    rdp = r0 % MAX_SAMPLES
    off = (rdp - index) % MAX_SAMPLES
    in_acts = off < chunk
    src_acts = jnp.minimum(d * chunk + off, DP * chunk - CH) + ACTS_FLAG
    src = jnp.where(in_acts, src_acts, r0).astype(jnp.int32)
    return jnp.broadcast_to(src[:, None], (NBLK, 16))


def _copy_body(acts_hbm, cache_hbm, tbl_hbm, out_hbm, tbl_v,
               b0, b1, b2, rs0, rs1, rs2, ws0, ws1, ws2):
    wid = lax.axis_index("c") * 16 + lax.axis_index("s")
    base = pl.multiple_of(wid * ROWS_PER_W, CH)
    pltpu.sync_copy(tbl_hbm.at[pl.ds(pl.multiple_of(wid * BPW, BPW), BPW)],
                    tbl_v)
    bufs = (b0, b1, b2)
    rsems = (rs0, rs1, rs2)
    wsems = (ws0, ws1, ws2)

    def start_read(i, buf, rsem):
        s = tbl_v[i][0]
        is_acts = s >= ACTS_FLAG

        @pl.when(is_acts)
        def _():
            pltpu.async_copy(
                acts_hbm.at[pl.ds(pl.multiple_of(s - ACTS_FLAG, 8), CH)],
                buf, rsem)

        @pl.when(jnp.logical_not(is_acts))
        def _():
            pltpu.async_copy(cache_hbm.at[pl.ds(pl.multiple_of(s, 8), CH)],
                             buf, rsem)

    def wait_read(buf, rsem):
        pltpu.make_async_copy(cache_hbm.at[pl.ds(0, CH)], buf, rsem).wait()

    def wait_write(buf, wsem):
        pltpu.make_async_copy(buf, out_hbm.at[pl.ds(base, CH)], wsem).wait()

    start_read(0, bufs[0], rsems[0])

    def body(i, _):
        for p in range(3):

            @pl.when((i % 3) == p)
            def _():
                pn = (p + 1) % 3

                @pl.when(jnp.logical_and(i + 1 < BPW, i >= 2))
                def _():
                    wait_write(bufs[pn], wsems[pn])

                @pl.when(i + 1 < BPW)
                def _():
                    start_read(i + 1, bufs[pn], rsems[pn])

                wait_read(bufs[p], rsems[p])
                pltpu.async_copy(
                    bufs[p],
                    out_hbm.at[pl.ds(pl.multiple_of(base + i * CH, CH), CH)],
                    wsems[p])

        return 0

    lax.fori_loop(0, BPW, body, 0)
    for p in range(3):
        wait_write(bufs[p], wsems[p])


def kernel(activations, cache, n_valid, index):
    dp, max_samples, d = cache.shape
    acts = activations.reshape((dp, -1, d))
    chunk = acts.shape[1]
    new_n_valid = jnp.minimum(jnp.asarray(n_valid) + chunk, max_samples)
    new_index = (jnp.asarray(index) + chunk) % max_samples

    acts_flat = activations.astype(cache.dtype)
    cache_flat = cache.reshape((dp * max_samples, d))
    tbl = _build_table(jnp.asarray(index, dtype=jnp.int32), chunk)

    mesh = plsc.VectorSubcoreMesh(core_axis_name="c", subcore_axis_name="s")
    out_flat = pl.kernel(
        _copy_body,
        mesh=mesh,
        out_type=jax.ShapeDtypeStruct((dp * max_samples, d), cache.dtype),
        scratch_types=[
            pltpu.VMEM((BPW, 16), jnp.int32),
            pltpu.VMEM((CH, N_DIM), jnp.float32),
            pltpu.VMEM((CH, N_DIM), jnp.float32),
            pltpu.VMEM((CH, N_DIM), jnp.float32),
            pltpu.SemaphoreType.DMA,
            pltpu.SemaphoreType.DMA,
            pltpu.SemaphoreType.DMA,
            pltpu.SemaphoreType.DMA,
            pltpu.SemaphoreType.DMA,
            pltpu.SemaphoreType.DMA,
        ],
    )(acts_flat, cache_flat, tbl)

    new_cache = out_flat.reshape((dp, max_samples, d))
    return (new_cache, new_n_valid, new_index)
